# baseline (speedup 1.0000x reference)
import jax
import jax.numpy as jnp
from jax import lax
from jax.experimental import pallas as pl
from jax.experimental.pallas import tpu as pltpu
from jax.experimental.pallas import tpu_sc as plsc

BATCH = 16384
DIM = 64
NW = 32
ROWS = BATCH // NW
C = 16
NCHUNK = ROWS // C
NBUF = 4
LANES = 16


def _transe_kernel(h_hbm, t_hbm, r_hbm, ent_hbm, rel_hbm, o_hbm, *refs):
    ivh, ivt, ivr = refs[0:3]
    prs = refs[3:3 + NBUF]
    hbufs = refs[3 + NBUF:3 + 2 * NBUF]
    tbufs = refs[3 + 2 * NBUF:3 + 3 * NBUF]
    rbufs = refs[3 + 3 * NBUF:3 + 4 * NBUF]
    obufs = refs[3 + 4 * NBUF:3 + 5 * NBUF]
    sems = refs[3 + 5 * NBUF:3 + 6 * NBUF]
    osems = refs[3 + 6 * NBUF:3 + 7 * NBUF]

    wid = lax.axis_index("core") * 16 + lax.axis_index("subcore")
    base = wid * ROWS

    pltpu.sync_copy(h_hbm.at[pl.ds(base, ROWS)], ivh)
    pltpu.sync_copy(t_hbm.at[pl.ds(base, ROWS)], ivt)
    pltpu.sync_copy(r_hbm.at[pl.ds(base, ROWS)], ivr)

    def issue(g, i):
        pr, hbuf, tbuf, rbuf, sem = prs[i], hbufs[i], tbufs[i], rbufs[i], sems[i]
        pr[...] = lax.shift_right_logical(ivr[pl.ds(g * C, C)], 1)
        pltpu.async_copy(rel_hbm.at[pr], rbuf, sem)
        hv = ivh[pl.ds(g * C, C)]
        tv = ivt[pl.ds(g * C, C)]
        for w in range(C):
            ih = hv[w]
            it = tv[w]
            pltpu.async_copy(
                ent_hbm.at[pl.ds(lax.shift_right_logical(ih, 3), 1),
                           pl.ds(ih & 7, 1)],
                hbuf.at[pl.ds(w // 8, 1), pl.ds(w % 8, 1)], sem)
            pltpu.async_copy(
                ent_hbm.at[pl.ds(lax.shift_right_logical(it, 3), 1),
                           pl.ds(it & 7, 1)],
                tbuf.at[pl.ds(w // 8, 1), pl.ds(w % 8, 1)], sem)

    def drain(i):
        pr, hbuf, tbuf, rbuf, sem = prs[i], hbufs[i], tbufs[i], rbufs[i], sems[i]
        pltpu.make_async_copy(rel_hbm.at[pr], rbuf, sem).wait()
        pltpu.make_async_copy(ent_hbm.at[pl.ds(0, C // 8)], hbuf, sem).wait()
        pltpu.make_async_copy(ent_hbm.at[pl.ds(0, C // 8)], tbuf, sem).wait()

    def compute(g, i):
        hbuf, tbuf, rbuf, obuf, osem = hbufs[i], tbufs[i], rbufs[i], obufs[i], osems[i]
        pltpu.make_async_copy(obuf, o_hbm.at[pl.ds(base, C)], osem).wait()
        rv = ivr[pl.ds(g * C, C)] & 1
        for w in range(C):
            rc = rv[w] * DIM
            for j in range(DIM // LANES):
                jo = j * LANES
                s = pl.ds(jo, LANES)
                obuf.at[w, s][...] = (
                    hbuf.at[w // 8, w % 8, s][...]
                    + rbuf.at[w, pl.ds(rc + jo, LANES)][...]
                    - tbuf.at[w // 8, w % 8, s][...]
                )
        pltpu.async_copy(obuf, o_hbm.at[pl.ds(base + g * C, C)], osem)

    for i in range(NBUF):
        pltpu.async_copy(obufs[i], o_hbm.at[pl.ds(base + i * C, C)], osems[i])
        issue(i, i)

    @pl.loop(0, NCHUNK, step=NBUF)
    def _(g):
        for i in range(NBUF):
            drain(i)
            compute(g + i, i)

            @pl.when(g + NBUF + i < NCHUNK)
            def _():
                issue(g + NBUF + i, i)

    for i in range(NBUF):
        pltpu.make_async_copy(obufs[i], o_hbm.at[pl.ds(base, C)],
                              osems[i]).wait()


@jax.jit
def kernel(h_list, t_list, r_list, ent_embeddings, rel_embeddings):
    n_ent, dim = ent_embeddings.shape
    n_rel = rel_embeddings.shape[0]
    mesh = plsc.VectorSubcoreMesh(core_axis_name="core",
                                  subcore_axis_name="subcore")
    idxbuf = pltpu.VMEM((ROWS,), jnp.int32)
    pairbuf = pltpu.VMEM((C,), jnp.int32)
    rowbuf = pltpu.VMEM((C // 8, 8, DIM), ent_embeddings.dtype)
    relbuf = pltpu.VMEM((C, 2 * DIM), ent_embeddings.dtype)
    outbuf = pltpu.VMEM((C, DIM), ent_embeddings.dtype)
    run = pl.kernel(
        _transe_kernel,
        out_type=jax.ShapeDtypeStruct((BATCH, DIM), ent_embeddings.dtype),
        mesh=mesh,
        scratch_types=(
            [idxbuf] * 3
            + [pairbuf] * NBUF
            + [rowbuf] * (2 * NBUF)
            + [relbuf] * NBUF
            + [outbuf] * NBUF
            + [pltpu.SemaphoreType.DMA] * (2 * NBUF)
        ),
    )
    return run(
        h_list.astype(jnp.int32),
        t_list.astype(jnp.int32),
        r_list.astype(jnp.int32),
        ent_embeddings.reshape(n_ent // 8, 8, dim),
        rel_embeddings.reshape(n_rel // 2, 2 * dim),
    )


# speedup vs baseline: 1.0245x; 1.0245x over previous
import jax
import jax.numpy as jnp
from jax import lax
from jax.experimental import pallas as pl
from jax.experimental.pallas import tpu as pltpu
from jax.experimental.pallas import tpu_sc as plsc

BATCH = 16384
DIM = 64
NW = 32
ROWS = BATCH // NW
C = 16
NCHUNK = ROWS // C
LANES = 16


def _transe_kernel(h_hbm, t_hbm, r_hbm, ent_hbm, rel_hbm, o_hbm,
                   ivh, ivt, ivr,
                   pr0, pr1,
                   hbuf0, hbuf1, tbuf0, tbuf1, rbuf0, rbuf1,
                   obuf0, obuf1,
                   sem0, sem1, osem0, osem1):
    wid = lax.axis_index("core") * 16 + lax.axis_index("subcore")
    base = wid * ROWS

    pltpu.sync_copy(h_hbm.at[pl.ds(base, ROWS)], ivh)
    pltpu.sync_copy(t_hbm.at[pl.ds(base, ROWS)], ivt)
    pltpu.sync_copy(r_hbm.at[pl.ds(base, ROWS)], ivr)

    def issue(g, pr, hbuf, tbuf, rbuf, sem):
        pr[...] = lax.shift_right_logical(ivr[pl.ds(g * C, C)], 1)
        pltpu.async_copy(rel_hbm.at[pr], rbuf, sem)
        for k in range(C // LANES):
            hv = ivh[pl.ds(g * C + k * LANES, LANES)]
            tv = ivt[pl.ds(g * C + k * LANES, LANES)]
            for u in range(LANES):
                w = k * LANES + u
                ih = hv[u]
                it = tv[u]
                pltpu.async_copy(
                    ent_hbm.at[pl.ds(lax.shift_right_logical(ih, 3), 1),
                               pl.ds(ih & 7, 1)],
                    hbuf.at[pl.ds(w // 8, 1), pl.ds(w % 8, 1)], sem)
                pltpu.async_copy(
                    ent_hbm.at[pl.ds(lax.shift_right_logical(it, 3), 1),
                               pl.ds(it & 7, 1)],
                    tbuf.at[pl.ds(w // 8, 1), pl.ds(w % 8, 1)], sem)

    def drain(pr, hbuf, tbuf, rbuf, sem):
        pltpu.make_async_copy(rel_hbm.at[pr], rbuf, sem).wait()
        pltpu.make_async_copy(ent_hbm.at[pl.ds(0, C // 8)], hbuf, sem).wait()
        pltpu.make_async_copy(ent_hbm.at[pl.ds(0, C // 8)], tbuf, sem).wait()

    def compute(g, hbuf, tbuf, rbuf, obuf, osem):
        pltpu.make_async_copy(obuf, o_hbm.at[pl.ds(base, C)], osem).wait()
        for k in range(C // LANES):
            rv = ivr[pl.ds(g * C + k * LANES, LANES)] & 1
            for u in range(LANES):
                w = k * LANES + u
                rc = rv[u] * DIM
                for j in range(DIM // LANES):
                    jo = j * LANES
                    s = pl.ds(jo, LANES)
                    obuf.at[w, s][...] = (
                        hbuf.at[w // 8, w % 8, s][...]
                        + rbuf.at[w, pl.ds(rc + jo, LANES)][...]
                        - tbuf.at[w // 8, w % 8, s][...]
                    )
        pltpu.async_copy(obuf, o_hbm.at[pl.ds(base + g * C, C)], osem)

    pltpu.async_copy(obuf0, o_hbm.at[pl.ds(base, C)], osem0)
    pltpu.async_copy(obuf1, o_hbm.at[pl.ds(base + C, C)], osem1)
    issue(0, pr0, hbuf0, tbuf0, rbuf0, sem0)

    @pl.loop(0, NCHUNK, step=2)
    def _(g):
        issue(g + 1, pr1, hbuf1, tbuf1, rbuf1, sem1)
        drain(pr0, hbuf0, tbuf0, rbuf0, sem0)
        compute(g, hbuf0, tbuf0, rbuf0, obuf0, osem0)

        @pl.when(g + 2 < NCHUNK)
        def _():
            issue(g + 2, pr0, hbuf0, tbuf0, rbuf0, sem0)

        drain(pr1, hbuf1, tbuf1, rbuf1, sem1)
        compute(g + 1, hbuf1, tbuf1, rbuf1, obuf1, osem1)

    pltpu.make_async_copy(obuf0, o_hbm.at[pl.ds(base, C)], osem0).wait()
    pltpu.make_async_copy(obuf1, o_hbm.at[pl.ds(base, C)], osem1).wait()


@jax.jit
def kernel(h_list, t_list, r_list, ent_embeddings, rel_embeddings):
    n_ent, dim = ent_embeddings.shape
    n_rel = rel_embeddings.shape[0]
    mesh = plsc.VectorSubcoreMesh(core_axis_name="core",
                                  subcore_axis_name="subcore")
    idxbuf = pltpu.VMEM((ROWS,), jnp.int32)
    pairbuf = pltpu.VMEM((C,), jnp.int32)
    rowbuf = pltpu.VMEM((C // 8, 8, DIM), ent_embeddings.dtype)
    relbuf = pltpu.VMEM((C, 2 * DIM), ent_embeddings.dtype)
    outbuf = pltpu.VMEM((C, DIM), ent_embeddings.dtype)
    run = pl.kernel(
        _transe_kernel,
        out_type=jax.ShapeDtypeStruct((BATCH, DIM), ent_embeddings.dtype),
        mesh=mesh,
        scratch_types=[
            idxbuf, idxbuf, idxbuf,
            pairbuf, pairbuf,
            rowbuf, rowbuf, rowbuf, rowbuf, relbuf, relbuf,
            outbuf, outbuf,
            pltpu.SemaphoreType.DMA,
            pltpu.SemaphoreType.DMA,
            pltpu.SemaphoreType.DMA,
            pltpu.SemaphoreType.DMA,
        ],
    )
    return run(
        h_list.astype(jnp.int32),
        t_list.astype(jnp.int32),
        r_list.astype(jnp.int32),
        ent_embeddings.reshape(n_ent // 8, 8, dim),
        rel_embeddings.reshape(n_rel // 2, 2 * dim),
    )
